# Initial kernel scaffold; baseline (speedup 1.0000x reference)
import jax
import jax.numpy as jnp
from jax.experimental import pallas as pl


def kernel(x, edge_index, batch, params):
    raise NotImplementedError("write your pallas kernel here")


# speedup vs baseline: 1.0714x; 1.0714x over previous
import functools

import jax
import jax.numpy as jnp
from jax.experimental import pallas as pl
from jax.experimental.pallas import tpu as pltpu

N = 10000
F_IN = 128
HID = 128
HEADS = 8
CH = 16
G = 64


def _gelu(v):
    return 0.5 * v * (1.0 + jax.lax.erf(v * 0.7071067811865476))


def _emb_body(x_ref, w_ref, b_ref, gamma_ref, beta_ref, o_ref):
    y = jnp.dot(x_ref[...], w_ref[...], preferred_element_type=jnp.float32)
    y = _gelu(y + b_ref[...][None, :])
    mean = jnp.mean(y, axis=0)
    var = jnp.mean(jnp.square(y), axis=0) - jnp.square(mean)
    o_ref[...] = gamma_ref[...][None, :] * (y - mean[None, :]) * jax.lax.rsqrt(
        var + 1e-5) + beta_ref[...][None, :]


def _emb(x, w, b, gamma, beta):
    return pl.pallas_call(
        _emb_body,
        out_shape=jax.ShapeDtypeStruct((N, HID), jnp.float32),
    )(x, w, b, gamma, beta)


def kernel(x, edge_index, batch, params):
    p = params
    h = _emb(x, p['emb_W'], p['emb_b'], p['emb_gamma'], p['emb_beta'])

    n = x.shape[0]
    sl = jnp.arange(n, dtype=edge_index.dtype)
    ei = jnp.concatenate([edge_index, jnp.stack([sl, sl], axis=0)], axis=1)
    src, dst = ei[0], ei[1]
    for lp in p['gat']:
        res = h
        xw = (h @ lp['W']).reshape(n, HEADS, CH)
        a_src = (xw * lp['att_src'][None]).sum(-1)
        a_dst = (xw * lp['att_dst'][None]).sum(-1)
        alpha = a_src[src] + a_dst[dst]
        alpha = jax.nn.leaky_relu(alpha, 0.2)
        ea = jnp.exp(alpha)
        denom = jax.ops.segment_sum(ea, dst, num_segments=n)
        num = jax.ops.segment_sum(xw[src] * ea[:, :, None], dst, num_segments=n)
        out = (num / (denom[:, :, None] + 1e-16)).reshape(n, HEADS * CH) + lp['bias']
        mean = out.mean(axis=0)
        var = out.var(axis=0)
        out = lp['gamma'] * (out - mean) * jax.lax.rsqrt(var + 1e-5) + lp['beta']
        h = _gelu(out) + res
    gate = _gelu(h @ p['gate_W1'] + p['gate_b1']) @ p['gate_W2'] + p['gate_b2']
    gate = gate[:, 0]
    eg = jnp.exp(gate - jnp.max(gate))
    gden = jax.ops.segment_sum(eg, batch, num_segments=G)
    attn = eg / (gden[batch] + 1e-16)
    pooled = jax.ops.segment_sum(attn[:, None] * h, batch, num_segments=G)
    o = _gelu(pooled @ p['out_W1'] + p['out_b1']) @ p['out_W2'] + p['out_b2']
    return o


# speedup vs baseline: 6.0535x; 5.6500x over previous
import functools

import jax
import jax.numpy as jnp
from jax import lax
from jax.experimental import pallas as pl
from jax.experimental.pallas import tpu as pltpu
from jax.experimental.pallas import tpu_sc as plsc

N = 10000
F_IN = 128
HID = 128
HEADS = 8
CH = 16
G = 64

NC = 2
NS = 16
C = 128
E2 = 320000 + N
K = -(-E2 // (NS * C))
E2P = NS * C * K
EPW = E2P // NS
NPAD = 10240
RPS = NPAD // NS
B = 20
W = 4 * B
NT = 2 * N + 8


def _gelu(v):
    return 0.5 * v * (1.0 + lax.erf(v * 0.7071067811865476))


def _bn(v, gamma, beta):
    mean = jnp.mean(v, axis=0)
    var = jnp.mean(jnp.square(v), axis=0) - jnp.square(mean)
    return gamma[None, :] * (v - mean[None, :]) * lax.rsqrt(var + 1e-5) + beta[None, :]


def _sc_body(xw_hbm, ts_hbm, td_hbm, srcg_hbm, dstg_hbm, dst_hbm, z_hbm,
             acc_out,
             srcg_v, dstg_v, dst_v, va_v, vb_v, xw_v, msg_v,
             acc,
             sem_a, sem_b, sem_x):
    c = lax.axis_index("c")
    s = lax.axis_index("s")
    row0 = s * RPS

    pltpu.sync_copy(z_hbm.at[pl.ds(row0, RPS)], acc.at[pl.ds(row0, RPS)])
    plsc.subcore_barrier()

    def _chunk(k, _):
        base = s * EPW + k * C
        pltpu.sync_copy(srcg_hbm.at[pl.ds(c * E2P + base, C)], srcg_v)
        pltpu.sync_copy(dstg_hbm.at[pl.ds(c * E2P + base, C)], dstg_v)
        pltpu.sync_copy(dst_hbm.at[pl.ds(base, C)], dst_v)
        ca = pltpu.async_copy(ts_hbm.at[srcg_v], va_v, sem_a)
        cb = pltpu.async_copy(td_hbm.at[dstg_v], vb_v, sem_b)
        cx = pltpu.async_copy(xw_hbm.at[srcg_v], xw_v, sem_x)
        ca.wait()
        cb.wait()
        cx.wait()

        def _edge(e, _):
            for j in range(W // CH):
                sl = pl.ds(j * CH, CH)
                a = va_v[e, sl] + vb_v[e, sl]
                ea = jnp.exp(jnp.maximum(a, 0.2 * a))
                msg_v[e, sl] = xw_v[e, sl] * ea
            return 0

        lax.fori_loop(0, C, _edge, 0)
        pltpu.sync_copy(msg_v, acc.at[dst_v], add=True)
        return 0

    lax.fori_loop(0, K, _chunk, 0)

    plsc.subcore_barrier()

    pltpu.sync_copy(acc.at[pl.ds(row0, RPS)],
                    acc_out.at[pl.ds(c * NPAD + row0, RPS)])


@functools.cache
def _sc_edge_fn():
    return functools.partial(
        pl.kernel,
        out_type=jax.ShapeDtypeStruct((NC * NPAD, W), jnp.float32),
        mesh=plsc.VectorSubcoreMesh(core_axis_name="c", subcore_axis_name="s",
                                    num_cores=NC, num_subcores=NS),
        scratch_types=[
            pltpu.VMEM((C,), jnp.int32),
            pltpu.VMEM((C,), jnp.int32),
            pltpu.VMEM((C,), jnp.int32),
            pltpu.VMEM((C, HID), jnp.float32),
            pltpu.VMEM((C, HID), jnp.float32),
            pltpu.VMEM((C, HID), jnp.float32),
            pltpu.VMEM((C, W), jnp.float32),
            pltpu.VMEM_SHARED((NPAD, W), jnp.float32),
            pltpu.SemaphoreType.DMA,
            pltpu.SemaphoreType.DMA,
            pltpu.SemaphoreType.DMA,
        ],
    )(_sc_body)


def _emb_body(x_ref, w_ref, b_ref, g_ref, be_ref, h_ref):
    y = jnp.dot(x_ref[...], w_ref[...], preferred_element_type=jnp.float32)
    y = _gelu(y + b_ref[...][None, :])
    h_ref[...] = _bn(y, g_ref[...], be_ref[...])


def _emb(x, p):
    return pl.pallas_call(
        _emb_body,
        out_shape=jax.ShapeDtypeStruct((N, HID), jnp.float32),
    )(x, p['emb_W'], p['emb_b'], p['emb_gamma'], p['emb_beta'])


def _proj_body(h_ref, w_ref, xm_ref, u_ref, as_ref, ad_ref,
               xw_ref, ts_ref, td_ref):
    xw = jnp.dot(h_ref[...], w_ref[...], preferred_element_type=jnp.float32)
    pad = jnp.zeros((8, 2 * HID), jnp.float32)
    xt = jnp.dot(xw, xm_ref[...], preferred_element_type=jnp.float32) + u_ref[...][None, :]
    xt = jnp.concatenate([xt, pad], axis=0)
    ts = jnp.concatenate(
        [jnp.dot(xw, as_ref[...], preferred_element_type=jnp.float32), pad], axis=0)
    td = jnp.concatenate(
        [jnp.dot(xw, ad_ref[...], preferred_element_type=jnp.float32), pad], axis=0)
    xw_ref[...] = jnp.concatenate([xt[:NT // 2, :HID], xt[:NT // 2, HID:]], axis=0)
    ts_ref[...] = jnp.concatenate([ts[:NT // 2, :HID], ts[:NT // 2, HID:]], axis=0)
    td_ref[...] = jnp.concatenate([td[:NT // 2, :HID], td[:NT // 2, HID:]], axis=0)


def _proj(h, w, xm, u, as_, ad_):
    return pl.pallas_call(
        _proj_body,
        out_shape=(
            jax.ShapeDtypeStruct((NT, HID), jnp.float32),
            jax.ShapeDtypeStruct((NT, HID), jnp.float32),
            jax.ShapeDtypeStruct((NT, HID), jnp.float32),
        ),
    )(h, w, xm, u, as_, ad_)


def _combine_body(acc_ref, bias_ref, g_ref, be_ref, res_ref, h_ref):
    outs = []
    for c in range(NC):
        for j in range(HEADS // NC):
            num = acc_ref[c, :N, j * B:j * B + CH]
            den = acc_ref[c, :N, j * B + CH]
            outs.append(num / (den[:, None] + 1e-16))
    out = jnp.concatenate(outs, axis=1) + bias_ref[...][None, :]
    out = _bn(out, g_ref[...], be_ref[...])
    h_ref[...] = _gelu(out) + res_ref[...]


def _combine(acc, lp, res):
    return pl.pallas_call(
        _combine_body,
        out_shape=jax.ShapeDtypeStruct((N, HID), jnp.float32),
    )(acc, lp['bias'], lp['gamma'], lp['beta'], res)


def _pool_body(h_ref, batch_ref, gw1_ref, gb1_ref, gw2_ref, gb2_ref,
               ow1_ref, ob1_ref, ow2_ref, ob2_ref, o_ref):
    h = h_ref[...]
    g1 = _gelu(jnp.dot(h, gw1_ref[...], preferred_element_type=jnp.float32)
               + gb1_ref[...][None, :])
    gate = jnp.sum(g1 * gw2_ref[...][None, :, 0], axis=1, keepdims=True) + gb2_ref[...][None, :]
    eg = jnp.exp(gate - jnp.max(gate))
    onehot = (batch_ref[...] ==
              lax.broadcasted_iota(jnp.int32, (G, N), 0)).astype(jnp.float32)
    gden = jnp.dot(onehot, eg, preferred_element_type=jnp.float32)
    pnum = jnp.dot(onehot, eg * h, preferred_element_type=jnp.float32)
    pooled = pnum / (gden + 1e-16)
    o1 = _gelu(jnp.dot(pooled, ow1_ref[...], preferred_element_type=jnp.float32)
               + ob1_ref[...][None, :])
    o_ref[...] = (jnp.sum(o1 * ow2_ref[...][None, :, 0], axis=1, keepdims=True)
                  + ob2_ref[...][None, :])


def _pool(h, batch, p):
    return pl.pallas_call(
        _pool_body,
        out_shape=jax.ShapeDtypeStruct((G, 1), jnp.float32),
    )(h, batch,
      p['gate_W1'], p['gate_b1'], p['gate_W2'], p['gate_b2'],
      p['out_W1'], p['out_b1'], p['out_W2'], p['out_b2'])


def _expand_matrices(a_src, a_dst):
    chan = jnp.arange(HID, dtype=jnp.int32)
    head = chan // CH
    ch = chan % CH
    outl = jnp.arange(2 * HID, dtype=jnp.int32)
    core = outl // HID
    inb = outl % HID
    valid = inb < W
    o_head = core * 4 + jnp.minimum(inb, W - 1) // B
    o_off = inb % B
    same_head = valid[None, :] & (head[:, None] == o_head[None, :])
    xm = (same_head & (ch[:, None] == o_off[None, :])).astype(jnp.float32)
    u = (valid & (o_off == CH)).astype(jnp.float32)
    asrc = jnp.where(same_head, a_src.reshape(-1)[:, None], 0.0)
    adst = jnp.where(same_head, a_dst.reshape(-1)[:, None], 0.0)
    return xm, u, asrc.astype(jnp.float32), adst.astype(jnp.float32)


def _xw_body(h_ref, w_ref, o_ref):
    o_ref[...] = jnp.dot(h_ref[...], w_ref[...], preferred_element_type=jnp.float32)


def _xw(h, w):
    return pl.pallas_call(
        _xw_body, out_shape=jax.ShapeDtypeStruct((N, HID), jnp.float32))(h, w)


def _combine2_body(num_ref, den_ref, bias_ref, g_ref, be_ref, res_ref, h_ref):
    out = num_ref[...] / (den_ref[...] + 1e-16) + bias_ref[...][None, :]
    out = _bn(out, g_ref[...], be_ref[...])
    h_ref[...] = _gelu(out) + res_ref[...]


def _combine2(num, den, lp, res):
    return pl.pallas_call(
        _combine2_body,
        out_shape=jax.ShapeDtypeStruct((N, HID), jnp.float32),
    )(num, den, lp['bias'], lp['gamma'], lp['beta'], res)


def kernel(x, edge_index, batch, params):
    p = params
    sl = jnp.arange(N, dtype=edge_index.dtype)
    src = jnp.concatenate([edge_index[0], sl])
    dst = jnp.concatenate([edge_index[1], sl])

    h = _emb(x, p)
    for l in range(4):
        lp = p['gat'][l]
        xw = _xw(h, lp['W'])
        xr = xw.reshape(N, HEADS, CH)
        a_src = (xr * lp['att_src'][None]).sum(-1)
        a_dst = (xr * lp['att_dst'][None]).sum(-1)
        alpha = a_src[src] + a_dst[dst]
        alpha = jnp.maximum(alpha, 0.2 * alpha)
        ea = jnp.exp(alpha)
        den = jax.ops.segment_sum(ea, dst, num_segments=N)
        msg = (xr[src] * ea[:, :, None]).reshape(-1, HID)
        num = jax.ops.segment_sum(msg, dst, num_segments=N)
        h = _combine2(num, jnp.repeat(den, CH, axis=1), lp, h)
    return _pool(h, batch[None, :], p)
